# baseline (speedup 1.0000x reference)
import functools

import jax
import jax.numpy as jnp
from jax.experimental import pallas as pl

_PP = 34 * 34 * 34
_SH, _SD = 34, 34 * 34
_TM = 2048
_NT = 19
_NQ = _NT * _TM
_NU = (_NT + 2) * _TM
_MARGIN = 857
_OFFS = tuple(kd * _SD + kh * _SH + kw
              for kd in range(3) for kh in range(3) for kw in range(3))


def _mid_body(u_ref, w_ref, b_ref, mask_ref, out_ref):
    m = pl.program_id(0)
    edge = (m == 0) | (m == _NT + 1)

    @pl.when(edge)
    def _():
        out_ref[:] = jnp.zeros_like(out_ref)

    @pl.when(jnp.logical_not(edge))
    def _():
        q0 = (m - 1) * _TM
        acc = jnp.zeros((_TM, out_ref.shape[1]), jnp.float32)
        for t, off in enumerate(_OFFS):
            a = u_ref[pl.ds(q0 + off + _MARGIN, _TM), :]
            acc += jnp.dot(a, w_ref[t], preferred_element_type=jnp.float32)
        y = jnp.maximum(acc + b_ref[:], 0.0)
        out_ref[:] = y * mask_ref[:]


def _last_body(u_ref, w_ref, b_ref, out_ref):
    q0 = pl.program_id(0) * _TM
    acc = jnp.zeros((_TM, out_ref.shape[1]), jnp.float32)
    for t, off in enumerate(_OFFS):
        a = u_ref[pl.ds(q0 + off + _MARGIN, _TM), :]
        acc += jnp.dot(a, w_ref[t], preferred_element_type=jnp.float32)
    out_ref[:] = acc + b_ref[:]


def _mid_layer(u, wt, b, mask):
    co = wt.shape[2]
    return pl.pallas_call(
        _mid_body,
        grid=(_NT + 2,),
        in_specs=[
            pl.BlockSpec((_NU, 64), lambda m: (0, 0)),
            pl.BlockSpec(wt.shape, lambda m: (0, 0, 0)),
            pl.BlockSpec((1, co), lambda m: (0, 0)),
            pl.BlockSpec((_TM, 1), lambda m: (jnp.clip(m - 1, 0, _NT - 1), 0)),
        ],
        out_specs=pl.BlockSpec((_TM, 64), lambda m: (m, 0)),
        out_shape=jax.ShapeDtypeStruct((_NU, 64), jnp.float32),
    )(u, wt, b, mask)


def _last_layer(u, wt, b):
    co = wt.shape[2]
    return pl.pallas_call(
        _last_body,
        grid=(_NT,),
        in_specs=[
            pl.BlockSpec((_NU, 64), lambda m: (0, 0)),
            pl.BlockSpec(wt.shape, lambda m: (0, 0, 0)),
            pl.BlockSpec((1, co), lambda m: (0, 0)),
        ],
        out_specs=pl.BlockSpec((_TM, co), lambda m: (m, 0)),
        out_shape=jax.ShapeDtypeStruct((_NQ, co), jnp.float32),
    )(u, wt, b)


def _prep_w(w, ci_pad):
    co, ci = w.shape[0], w.shape[1]
    wt = jnp.transpose(w, (2, 3, 4, 1, 0)).reshape(27, ci, co)
    if ci < ci_pad:
        wt = jnp.pad(wt, ((0, 0), (0, ci_pad - ci), (0, 0)))
    return wt


@functools.partial(jax.jit, static_argnums=())
def kernel(x, W1, b1, W2, b2, W3, b3, W4, b4, W5, b5):
    xc = jnp.transpose(x[0], (1, 2, 3, 0))
    xp = jnp.pad(xc, ((1, 1), (1, 1), (1, 1), (0, 64 - 36)))
    u0 = jnp.pad(xp.reshape(_PP, 64),
                 ((_MARGIN, _NU - _PP - _MARGIN), (0, 0)))

    ones = jnp.pad(jnp.ones((32, 32, 32), jnp.float32),
                   ((1, 1), (1, 1), (1, 1))).reshape(_PP)
    maskq = jnp.pad(ones[1191:], ((0, _NQ - (_PP - 1191)),))[:, None]

    u = _mid_layer(u0, _prep_w(W1, 64), b1[None, :], maskq)
    u = _mid_layer(u, _prep_w(W2, 64), b2[None, :], maskq)
    u = _mid_layer(u, _prep_w(W3, 64), b3[None, :], maskq)
    u = _mid_layer(u, _prep_w(W4, 64), b4[None, :], maskq)
    y = _last_layer(u, _prep_w(W5, 64), b5[None, :])

    y = jnp.pad(y, ((0, _PP - _NQ), (0, 0))) if _NQ < _PP else y[:_PP]
    y = y.reshape(34, 34, 34, 54)[:32, :32, :32, :]
    return y.reshape(1, 32 * 32 * 32 * 9, 6)


# speedup vs baseline: 1.0307x; 1.0307x over previous
import functools

import jax
import jax.numpy as jnp
from jax.experimental import pallas as pl
from jax.experimental.pallas import tpu as pltpu

_SD, _SH = 1360, 40
_PP = 34 * 34 * 40
_TM = 2048
_NT = 22
_NQ = _NT * _TM
_V0 = 16
_NU = 47888
_SB = 1408
_EXT = 2080
_SUB = 208
_SUBL = 256
_OFF9 = tuple(kd * _SD + kh * _SH for kd in range(3) for kh in range(3))


def _tap_acc(u_ref, w_ref, v0, rows):
    acc = jnp.zeros((rows, w_ref.shape[2]), jnp.float32)
    for t, off in enumerate(_OFF9):
        a = u_ref[pl.ds(v0 + off, rows), :]
        acc += jnp.dot(a, w_ref[t], preferred_element_type=jnp.float32)
    return acc


def _mid_body(u_ref, w_ref, b_ref, m0_ref, m1_ref, m2_ref, out_ref, ys_ref):
    m = pl.program_id(0)

    @pl.when(m == 0)
    def _():
        out_ref[pl.ds(0, _SB), :] = jnp.zeros((_SB, 192), jnp.bfloat16)
        hi = _NQ + _SB
        out_ref[pl.ds(hi, _NU - hi), :] = jnp.zeros((_NU - hi, 192), jnp.bfloat16)

    q0 = m * _TM
    for s in range(_EXT // _SUB):
        acc = _tap_acc(u_ref, w_ref, q0 + s * _SUB, _SUB)
        ys_ref[pl.ds(s * _SUB, _SUB), :] = jnp.maximum(acc + b_ref[:], 0.0)
    for j, mask in enumerate((m0_ref, m1_ref, m2_ref)):
        yj = ys_ref[pl.ds(j + 7, _TM), :]
        yj = (yj * mask[:]).astype(jnp.bfloat16)
        out_ref[pl.ds(q0 + _SB, _TM), 64 * j:64 * (j + 1)] = yj


def _last_body(u_ref, w_ref, b_ref, out_ref):
    q0 = pl.program_id(0) * _TM
    for s in range(_TM // _SUBL):
        acc = _tap_acc(u_ref, w_ref, q0 + s * _SUBL + _V0, _SUBL)
        out_ref[pl.ds(s * _SUBL, _SUBL), :] = acc + b_ref[:]


def _mid_layer(u, wg, b, masks):
    return pl.pallas_call(
        _mid_body,
        grid=(_NT,),
        in_specs=[
            pl.BlockSpec((_NU, 192), lambda m: (0, 0)),
            pl.BlockSpec(wg.shape, lambda m: (0, 0, 0)),
            pl.BlockSpec((1, 64), lambda m: (0, 0)),
            pl.BlockSpec((_TM, 1), lambda m: (m, 0)),
            pl.BlockSpec((_TM, 1), lambda m: (m, 0)),
            pl.BlockSpec((_TM, 1), lambda m: (m, 0)),
        ],
        out_specs=pl.BlockSpec((_NU, 192), lambda m: (0, 0)),
        out_shape=jax.ShapeDtypeStruct((_NU, 192), jnp.bfloat16),
        scratch_shapes=[pltpu.VMEM((_EXT, 64), jnp.float32)],
    )(u, wg, b, *masks)


def _last_layer(u, wg, b):
    co = wg.shape[2]
    return pl.pallas_call(
        _last_body,
        grid=(_NT,),
        in_specs=[
            pl.BlockSpec((_NU, 192), lambda m: (0, 0)),
            pl.BlockSpec(wg.shape, lambda m: (0, 0, 0)),
            pl.BlockSpec((1, co), lambda m: (0, 0)),
        ],
        out_specs=pl.BlockSpec((_TM, co), lambda m: (m, 0)),
        out_shape=jax.ShapeDtypeStruct((_NQ, co), jnp.float32),
    )(u, wg, b)


def _prep_w(w, ci_pad):
    co, ci = w.shape[0], w.shape[1]
    wt = jnp.transpose(w, (2, 3, 4, 1, 0))
    if ci < ci_pad:
        wt = jnp.pad(wt, ((0, 0), (0, 0), (0, 0), (0, ci_pad - ci), (0, 0)))
    return wt.reshape(9, 3 * ci_pad, co).astype(jnp.bfloat16)


def _pack_u(a_flat):
    blocks = []
    for j in range(3):
        lo = _V0 - j
        blocks.append(jnp.pad(a_flat, ((lo, _NU - _PP - lo), (0, 0))))
    return jnp.concatenate(blocks, axis=1).astype(jnp.bfloat16)


@functools.partial(jax.jit, static_argnums=())
def kernel(x, W1, b1, W2, b2, W3, b3, W4, b4, W5, b5):
    xc = jnp.transpose(x[0], (1, 2, 3, 0))
    xp = jnp.pad(xc, ((1, 1), (1, 1), (1, 7), (0, 64 - 36)))
    u = _pack_u(xp.reshape(_PP, 64))

    interior = jnp.pad(jnp.ones((32, 32, 32), jnp.float32),
                       ((1, 1), (1, 1), (1, 7))).reshape(_PP)
    interior = jnp.pad(interior, ((0, _NQ + 1392 + 3 - _PP),))
    masks = tuple(interior[1392 + j:1392 + j + _NQ][:, None] for j in range(3))

    for wi, bi in ((W1, b1), (W2, b2), (W3, b3), (W4, b4)):
        u = _mid_layer(u, _prep_w(wi, 64), bi[None, :], masks)
    y = _last_layer(u, _prep_w(W5, 64), b5[None, :])

    y = jnp.pad(y, ((0, _PP - _NQ), (0, 0)))
    y = y.reshape(34, 34, 40, 54)[:32, :32, :32, :]
    return y.reshape(1, 32 * 32 * 32 * 9, 6)


# speedup vs baseline: 1.1063x; 1.0733x over previous
import functools

import jax
import jax.numpy as jnp
from jax.experimental import pallas as pl
from jax.experimental.pallas import tpu as pltpu

_SD, _SH = 1360, 40
_PP = 34 * 34 * 40
_TM = 4096
_NT = 11
_NQ = _NT * _TM
_V0 = 16
_NU = 47920
_SB = 1408
_EXT = 4160
_SUB = 208
_SUBL = 256
_OFF9 = tuple(kd * _SD + kh * _SH for kd in range(3) for kh in range(3))


def _tap_acc(u_ref, w_ref, v0, rows):
    acc = jnp.zeros((rows, w_ref.shape[2]), jnp.float32)
    for t, off in enumerate(_OFF9):
        a = u_ref[pl.ds(v0 + off, rows), :]
        acc += jnp.dot(a, w_ref[t], preferred_element_type=jnp.float32)
    return acc


def _mid_body(u_ref, w_ref, b_ref, mask_ref, out_ref, ys_ref):
    m = pl.program_id(0)

    @pl.when(m == 0)
    def _():
        out_ref[pl.ds(0, _SB), :] = jnp.zeros((_SB, 192), jnp.bfloat16)
        hi = _NQ + _SB
        out_ref[pl.ds(hi, _NU - hi), :] = jnp.zeros((_NU - hi, 192), jnp.bfloat16)

    q0 = m * _TM
    for s in range(_EXT // _SUB):
        acc = _tap_acc(u_ref, w_ref, q0 + s * _SUB, _SUB)
        ys_ref[pl.ds(s * _SUB, _SUB), :] = jnp.maximum(acc + b_ref[:], 0.0)
    for j in range(3):
        yj = ys_ref[pl.ds(j + 7, _TM), :]
        yj = yj.astype(jnp.bfloat16) * mask_ref[:, j:j + 1]
        out_ref[pl.ds(q0 + _SB, _TM), 64 * j:64 * (j + 1)] = yj


def _last_body(u_ref, w_ref, b_ref, out_ref):
    q0 = pl.program_id(0) * _TM
    for s in range(_TM // _SUBL):
        acc = _tap_acc(u_ref, w_ref, q0 + s * _SUBL + _V0, _SUBL)
        out_ref[pl.ds(s * _SUBL, _SUBL), :] = acc + b_ref[:]


def _mid_layer(u, wg, b, mask3):
    return pl.pallas_call(
        _mid_body,
        grid=(_NT,),
        in_specs=[
            pl.BlockSpec((_NU, 192), lambda m: (0, 0)),
            pl.BlockSpec(wg.shape, lambda m: (0, 0, 0)),
            pl.BlockSpec((1, 64), lambda m: (0, 0)),
            pl.BlockSpec((_TM, 3), lambda m: (m, 0)),
        ],
        out_specs=pl.BlockSpec((_NU, 192), lambda m: (0, 0)),
        out_shape=jax.ShapeDtypeStruct((_NU, 192), jnp.bfloat16),
        scratch_shapes=[pltpu.VMEM((_EXT, 64), jnp.float32)],
    )(u, wg, b, mask3)


def _last_layer(u, wg, b):
    co = wg.shape[2]
    return pl.pallas_call(
        _last_body,
        grid=(_NT,),
        in_specs=[
            pl.BlockSpec((_NU, 192), lambda m: (0, 0)),
            pl.BlockSpec(wg.shape, lambda m: (0, 0, 0)),
            pl.BlockSpec((1, co), lambda m: (0, 0)),
        ],
        out_specs=pl.BlockSpec((_TM, co), lambda m: (m, 0)),
        out_shape=jax.ShapeDtypeStruct((_NQ, co), jnp.float32),
    )(u, wg, b)


def _prep_w(w, ci_pad):
    co, ci = w.shape[0], w.shape[1]
    wt = jnp.transpose(w, (2, 3, 4, 1, 0))
    if ci < ci_pad:
        wt = jnp.pad(wt, ((0, 0), (0, 0), (0, 0), (0, ci_pad - ci), (0, 0)))
    return wt.reshape(9, 3 * ci_pad, co).astype(jnp.bfloat16)


def _pack_u(a_flat):
    blocks = []
    for j in range(3):
        lo = _V0 - j
        blocks.append(jnp.pad(a_flat, ((lo, _NU - _PP - lo), (0, 0))))
    return jnp.concatenate(blocks, axis=1).astype(jnp.bfloat16)


@functools.partial(jax.jit, static_argnums=())
def kernel(x, W1, b1, W2, b2, W3, b3, W4, b4, W5, b5):
    xc = jnp.transpose(x[0], (1, 2, 3, 0))
    xp = jnp.pad(xc, ((1, 1), (1, 1), (1, 7), (0, 64 - 36)))
    u = _pack_u(xp.reshape(_PP, 64))

    interior = jnp.pad(jnp.ones((32, 32, 32), jnp.float32),
                       ((1, 1), (1, 1), (1, 7))).reshape(_PP)
    interior = jnp.pad(interior, ((0, _NQ + 1392 + 3 - _PP),))
    mask3 = jnp.stack([interior[1392 + j:1392 + j + _NQ] for j in range(3)],
                      axis=1).astype(jnp.bfloat16)

    for wi, bi in ((W1, b1), (W2, b2), (W3, b3), (W4, b4)):
        u = _mid_layer(u, _prep_w(wi, 64), bi[None, :], mask3)
    y = _last_layer(u, _prep_w(W5, 64), b5[None, :])

    y = jnp.pad(y, ((0, _PP - _NQ), (0, 0)))
    y = y.reshape(34, 34, 40, 54)[:32, :32, :32, :]
    return y.reshape(1, 32 * 32 * 32 * 9, 6)
